# Initial kernel scaffold; baseline (speedup 1.0000x reference)
import jax
import jax.numpy as jnp
from jax.experimental import pallas as pl


def kernel(x, edge_index, edge_weight, W1_0, b1_0, W2_0, b2_0, W1_1, b1_1, W2_1, b2_1):
    raise NotImplementedError("write your pallas kernel here")


# speedup vs baseline: 5.8160x; 5.8160x over previous
import functools

import jax
import jax.numpy as jnp
from jax import lax
from jax.experimental import pallas as pl
from jax.experimental.pallas import tpu as pltpu
from jax.experimental.pallas import tpu_sc as plsc

N = 10000
E = 320000
D = 128

NC = 2
NS = 16
NW = NC * NS
EPW = E // NW
K = 80
NB = EPW // K
RPT = 624
REM = N - NS * RPT
SB = 25
CH = SB * K
NSC = NB // SB

_mesh = plsc.VectorSubcoreMesh(core_axis_name="c", subcore_axis_name="s")


@functools.partial(
    pl.kernel,
    out_type=jax.ShapeDtypeStruct((2 * N, D), jnp.float32),
    mesh=_mesh,
    scratch_types=[
        pltpu.VMEM((CH,), jnp.int32),
        pltpu.VMEM((NB, K), jnp.int32),
        pltpu.VMEM((CH,), jnp.float32),
        pltpu.VMEM((K, D), jnp.float32),
        pltpu.VMEM_SHARED((N, D), jnp.float32),
    ],
)
def _side_sc(ego_hbm, col_hbm, row_hbm, w_hbm, out_hbm,
             colv, rowv, wv, rows, acc):
    c = lax.axis_index("c")
    s = lax.axis_index("s")
    wid = c * NS + s

    zero16 = jnp.zeros((16,), jnp.float32)

    def _zrow(i, carry):
        for j in range(D // 16):
            rows[i, pl.ds(j * 16, 16)] = zero16
        return carry

    lax.fori_loop(0, K, _zrow, 0)
    for q in range(RPT // K):
        pltpu.sync_copy(rows, acc.at[pl.ds(s * RPT + q * K, K)])
    pltpu.sync_copy(rows.at[pl.ds(0, RPT % K)],
                    acc.at[pl.ds(s * RPT + (RPT // K) * K, RPT % K)])

    @pl.when(s == NS - 1)
    def _zero_rem():
        pltpu.sync_copy(rows.at[pl.ds(0, REM)], acc.at[pl.ds(NS * RPT, REM)])

    plsc.subcore_barrier()

    pltpu.sync_copy(row_hbm.at[wid], rowv)

    def _chunk(u, carry):
        pltpu.sync_copy(col_hbm.at[pl.ds(wid * EPW + u * CH, CH)], colv)
        pltpu.sync_copy(w_hbm.at[pl.ds(wid * EPW + u * CH, CH)], wv)

        def _batch(b, carry1):
            pltpu.sync_copy(ego_hbm.at[colv.at[pl.ds(b * K, K)]], rows)

            def _scale(g, carry2):
                wvec16 = wv[pl.ds(b * K + g * 16, 16)]
                for e16 in range(16):
                    wb = wvec16[jnp.full((16,), e16, jnp.int32)]
                    e = g * 16 + e16
                    for j in range(D // 16):
                        rows[e, pl.ds(j * 16, 16)] = (
                            rows[e, pl.ds(j * 16, 16)] * wb)
                return carry2

            lax.fori_loop(0, K // 16, _scale, 0)
            pltpu.sync_copy(rows, acc.at[rowv.at[u * SB + b]], add=True)
            return carry1

        lax.fori_loop(0, SB, _batch, 0)
        return carry

    lax.fori_loop(0, NSC, _chunk, 0)
    plsc.subcore_barrier()

    pltpu.sync_copy(acc.at[pl.ds(s * RPT, RPT)],
                    out_hbm.at[pl.ds(c * N + s * RPT, RPT)])

    @pl.when(s == NS - 1)
    def _out_rem():
        pltpu.sync_copy(acc.at[pl.ds(NS * RPT, REM)],
                        out_hbm.at[pl.ds(c * N + NS * RPT, REM)])


BLK = 2000


def _dense_body(ego_ref, sa_ref, sb_ref, w1_ref, b1_ref, w2_ref, b2_ref,
                ego_out_ref, nrm_out_ref):
    ego = ego_ref[...]
    side = sa_ref[...] + sb_ref[...]
    h1 = jnp.dot(ego + side, w1_ref[...],
                 preferred_element_type=jnp.float32) + b1_ref[...]
    h1 = jnp.maximum(h1, 0.01 * h1)
    h2 = jnp.dot(ego * side, w2_ref[...],
                 preferred_element_type=jnp.float32) + b2_ref[...]
    h2 = jnp.maximum(h2, 0.01 * h2)
    e = h1 + h2
    ego_out_ref[...] = e
    ss = jnp.sum(e * e, axis=1, keepdims=True)
    nrm_out_ref[...] = e / jnp.maximum(jnp.sqrt(ss), 1e-12)


def _dense_tc(ego, side2, W1, b1, W2, b2):
    nblk = N // BLK
    return pl.pallas_call(
        _dense_body,
        grid=(nblk,),
        in_specs=[
            pl.BlockSpec((BLK, D), lambda i: (i, 0)),
            pl.BlockSpec((BLK, D), lambda i: (i, 0)),
            pl.BlockSpec((BLK, D), lambda i, _n=nblk: (i + _n, 0)),
            pl.BlockSpec((D, D), lambda i: (0, 0)),
            pl.BlockSpec((1, D), lambda i: (0, 0)),
            pl.BlockSpec((D, D), lambda i: (0, 0)),
            pl.BlockSpec((1, D), lambda i: (0, 0)),
        ],
        out_specs=[
            pl.BlockSpec((BLK, D), lambda i: (i, 0)),
            pl.BlockSpec((BLK, D), lambda i: (i, 0)),
        ],
        out_shape=[
            jax.ShapeDtypeStruct((N, D), jnp.float32),
            jax.ShapeDtypeStruct((N, D), jnp.float32),
        ],
    )(ego, side2, side2, W1, b1, W2, b2)


def kernel(x, edge_index, edge_weight, W1_0, b1_0, W2_0, b2_0,
           W1_1, b1_1, W2_1, b2_1):
    row = edge_index[0].astype(jnp.int32).reshape(NW, NB, K)
    col = edge_index[1].astype(jnp.int32)
    w = edge_weight.astype(jnp.float32)

    params = [(W1_0, b1_0.reshape(1, D), W2_0, b2_0.reshape(1, D)),
              (W1_1, b1_1.reshape(1, D), W2_1, b2_1.reshape(1, D))]
    ego = x
    outs = [x]
    for (W1, b1, W2, b2) in params:
        side2 = _side_sc(ego, col, row, w)
        ego, nrm = _dense_tc(ego, side2, W1, b1, W2, b2)
        outs.append(nrm)
    return jnp.concatenate(outs, axis=1)


# speedup vs baseline: 8.5425x; 1.4688x over previous
import functools

import jax
import jax.numpy as jnp
from jax import lax
from jax.experimental import pallas as pl
from jax.experimental.pallas import tpu as pltpu
from jax.experimental.pallas import tpu_sc as plsc

N = 10000
E = 320000
D = 128

NC = 2
NS = 16
NW = NC * NS
EPW = E // NW
K = 40
NB = EPW // K
RPT = 624
REM = N - NS * RPT
SB = 50
SBP = SB // 2
CH = SB * K
NSC = NB // SB

_mesh = plsc.VectorSubcoreMesh(core_axis_name="c", subcore_axis_name="s")


@functools.partial(
    pl.kernel,
    out_type=jax.ShapeDtypeStruct((2 * N, D), jnp.float32),
    mesh=_mesh,
    scratch_types=[
        pltpu.VMEM((CH,), jnp.int32),
        pltpu.VMEM((SB, K), jnp.int32),
        pltpu.VMEM((CH,), jnp.float32),
        pltpu.VMEM((K, D), jnp.float32),
        pltpu.VMEM((K, D), jnp.float32),
        pltpu.VMEM((K, D), jnp.float32),
        pltpu.VMEM((K, D), jnp.float32),
        pltpu.VMEM_SHARED((N, D), jnp.float32),
        pltpu.SemaphoreType.DMA,
        pltpu.SemaphoreType.DMA,
        pltpu.SemaphoreType.DMA,
        pltpu.SemaphoreType.DMA,
    ],
)
def _side_sc(ego_hbm, col_hbm, row_hbm, w_hbm, out_hbm,
             colv, rowv, wv, r0, r1, t0, t1, acc, g0, g1, s0, s1):
    c = lax.axis_index("c")
    s = lax.axis_index("s")
    wid = c * NS + s

    zero16 = jnp.zeros((16,), jnp.float32)

    def _zrow(i, carry):
        for j in range(D // 16):
            r0[i, pl.ds(j * 16, 16)] = zero16
        return carry

    lax.fori_loop(0, K, _zrow, 0)
    for q in range(RPT // K):
        pltpu.sync_copy(r0, acc.at[pl.ds(s * RPT + q * K, K)])
    pltpu.sync_copy(r0.at[pl.ds(0, RPT % K)],
                    acc.at[pl.ds(s * RPT + (RPT // K) * K, RPT % K)])

    @pl.when(s == NS - 1)
    def _zero_rem():
        pltpu.sync_copy(r0.at[pl.ds(0, REM)], acc.at[pl.ds(NS * RPT, REM)])

    plsc.subcore_barrier()

    def _gather(b, rbuf, sem):
        return pltpu.async_copy(
            ego_hbm.at[colv.at[pl.ds(b * K, K)]], rbuf, sem)

    def _gather_wait(b, rbuf, sem):
        pltpu.make_async_copy(
            ego_hbm.at[colv.at[pl.ds(b * K, K)]], rbuf, sem).wait()

    def _scatter(b, tbuf, sem):
        return pltpu.async_copy(tbuf, acc.at[rowv.at[b]], sem, add=True)

    def _scatter_wait(b, tbuf, sem):
        pltpu.make_async_copy(tbuf, acc.at[rowv.at[b]], sem).wait()

    def _scale(b, rbuf, tbuf):
        for g in range((K + 15) // 16):
            wvec16 = wv[pl.ds(b * K + g * 16, 16)]
            for e16 in range(min(16, K - g * 16)):
                wb = wvec16[jnp.full((16,), e16, jnp.int32)]
                e = g * 16 + e16
                for j in range(D // 16):
                    tbuf[e, pl.ds(j * 16, 16)] = (
                        rbuf[e, pl.ds(j * 16, 16)] * wb)

    def _chunk(u, carry):
        pltpu.sync_copy(col_hbm.at[pl.ds(wid * EPW + u * CH, CH)], colv)
        pltpu.sync_copy(w_hbm.at[pl.ds(wid * EPW + u * CH, CH)], wv)
        pltpu.sync_copy(row_hbm.at[wid, u], rowv)

        _gather(0, r0, g0)
        _gather(1, r1, g1)

        def _pair(p, carry1):
            b0 = 2 * p
            b1 = 2 * p + 1
            _gather_wait(b0, r0, g0)

            @pl.when(p > 0)
            def _w0():
                _scatter_wait(b0, t0, s0)

            _scale(b0, r0, t0)

            @pl.when(p < SBP - 1)
            def _g0():
                _gather(b0 + 2, r0, g0)

            _scatter(b0, t0, s0)

            _gather_wait(b1, r1, g1)

            @pl.when(p > 0)
            def _w1():
                _scatter_wait(b1, t1, s1)

            _scale(b1, r1, t1)

            @pl.when(p < SBP - 1)
            def _g1():
                _gather(b1 + 2, r1, g1)

            _scatter(b1, t1, s1)
            return carry1

        lax.fori_loop(0, SBP, _pair, 0)
        _scatter_wait(0, t0, s0)
        _scatter_wait(0, t1, s1)
        return carry

    lax.fori_loop(0, NSC, _chunk, 0)
    plsc.subcore_barrier()

    pltpu.sync_copy(acc.at[pl.ds(s * RPT, RPT)],
                    out_hbm.at[pl.ds(c * N + s * RPT, RPT)])

    @pl.when(s == NS - 1)
    def _out_rem():
        pltpu.sync_copy(acc.at[pl.ds(NS * RPT, REM)],
                        out_hbm.at[pl.ds(c * N + NS * RPT, REM)])


BLK = 2000


def _dense_body(ego_ref, sa_ref, sb_ref, w1_ref, b1_ref, w2_ref, b2_ref,
                ego_out_ref, nrm_out_ref):
    ego = ego_ref[...]
    side = sa_ref[...] + sb_ref[...]
    h1 = jnp.dot(ego + side, w1_ref[...],
                 preferred_element_type=jnp.float32) + b1_ref[...]
    h1 = jnp.maximum(h1, 0.01 * h1)
    h2 = jnp.dot(ego * side, w2_ref[...],
                 preferred_element_type=jnp.float32) + b2_ref[...]
    h2 = jnp.maximum(h2, 0.01 * h2)
    e = h1 + h2
    ego_out_ref[...] = e
    ss = jnp.sum(e * e, axis=1, keepdims=True)
    nrm_out_ref[...] = e / jnp.maximum(jnp.sqrt(ss), 1e-12)


def _dense_tc(ego, side2, W1, b1, W2, b2):
    nblk = N // BLK
    return pl.pallas_call(
        _dense_body,
        grid=(nblk,),
        in_specs=[
            pl.BlockSpec((BLK, D), lambda i: (i, 0)),
            pl.BlockSpec((BLK, D), lambda i: (i, 0)),
            pl.BlockSpec((BLK, D), lambda i, _n=nblk: (i + _n, 0)),
            pl.BlockSpec((D, D), lambda i: (0, 0)),
            pl.BlockSpec((1, D), lambda i: (0, 0)),
            pl.BlockSpec((D, D), lambda i: (0, 0)),
            pl.BlockSpec((1, D), lambda i: (0, 0)),
        ],
        out_specs=[
            pl.BlockSpec((BLK, D), lambda i: (i, 0)),
            pl.BlockSpec((BLK, D), lambda i: (i, 0)),
        ],
        out_shape=[
            jax.ShapeDtypeStruct((N, D), jnp.float32),
            jax.ShapeDtypeStruct((N, D), jnp.float32),
        ],
    )(ego, side2, side2, W1, b1, W2, b2)


def kernel(x, edge_index, edge_weight, W1_0, b1_0, W2_0, b2_0,
           W1_1, b1_1, W2_1, b2_1):
    row = edge_index[0].astype(jnp.int32).reshape(NW, NSC, SB, K)
    col = edge_index[1].astype(jnp.int32)
    w = edge_weight.astype(jnp.float32)

    params = [(W1_0, b1_0.reshape(1, D), W2_0, b2_0.reshape(1, D)),
              (W1_1, b1_1.reshape(1, D), W2_1, b2_1.reshape(1, D))]
    ego = x
    outs = [x]
    for (W1, b1, W2, b2) in params:
        side2 = _side_sc(ego, col, row, w)
        ego, nrm = _dense_tc(ego, side2, W1, b1, W2, b2)
        outs.append(nrm)
    return jnp.concatenate(outs, axis=1)


# speedup vs baseline: 9.1872x; 1.0755x over previous
import functools

import jax
import jax.numpy as jnp
from jax import lax
from jax.experimental import pallas as pl
from jax.experimental.pallas import tpu as pltpu
from jax.experimental.pallas import tpu_sc as plsc

N = 10000
E = 320000
D = 128

NC = 2
NS = 16
NW = NC * NS
EPW = E // NW
K = 40
NB = EPW // K
NPAIR = NB // 2
RPT = 624
REM = N - NS * RPT
SB = 10
CH = SB * K
NSC = NB // SB

_mesh = plsc.VectorSubcoreMesh(core_axis_name="c", subcore_axis_name="s")


@functools.partial(
    pl.kernel,
    out_type=jax.ShapeDtypeStruct((2 * N, D), jnp.float32),
    mesh=_mesh,
    scratch_types=[
        pltpu.VMEM((2 * CH,), jnp.int32),
        pltpu.VMEM((2, SB, K), jnp.int32),
        pltpu.VMEM((2 * CH,), jnp.float32),
        pltpu.VMEM((K, D), jnp.float32),
        pltpu.VMEM((K, D), jnp.float32),
        pltpu.VMEM((K, D), jnp.float32),
        pltpu.VMEM((K, D), jnp.float32),
        pltpu.VMEM_SHARED((N, D), jnp.float32),
        pltpu.SemaphoreType.DMA,
        pltpu.SemaphoreType.DMA,
        pltpu.SemaphoreType.DMA,
        pltpu.SemaphoreType.DMA,
        pltpu.SemaphoreType.DMA,
        pltpu.SemaphoreType.DMA,
    ],
)
def _side_sc(ego_hbm, col_hbm, row_hbm, w_hbm, out_hbm,
             colv, rowv, wv, r0, r1, t0, t1, acc,
             g0, g1, s0, s1, stg0, stg1):
    c = lax.axis_index("c")
    s = lax.axis_index("s")
    wid = c * NS + s

    zero16 = jnp.zeros((16,), jnp.float32)

    def _zrow(i, carry):
        for j in range(D // 16):
            r0[i, pl.ds(j * 16, 16)] = zero16
        return carry

    lax.fori_loop(0, K, _zrow, 0)
    for q in range(RPT // K):
        pltpu.sync_copy(r0, acc.at[pl.ds(s * RPT + q * K, K)])
    pltpu.sync_copy(r0.at[pl.ds(0, RPT % K)],
                    acc.at[pl.ds(s * RPT + (RPT // K) * K, RPT % K)])

    @pl.when(s == NS - 1)
    def _zero_rem():
        pltpu.sync_copy(r0.at[pl.ds(0, REM)], acc.at[pl.ds(NS * RPT, REM)])

    plsc.subcore_barrier()

    def _stage(u, slot, sem):
        pltpu.async_copy(col_hbm.at[pl.ds(wid * EPW + u * CH, CH)],
                         colv.at[pl.ds(slot * CH, CH)], sem)
        pltpu.async_copy(w_hbm.at[pl.ds(wid * EPW + u * CH, CH)],
                         wv.at[pl.ds(slot * CH, CH)], sem)
        pltpu.async_copy(row_hbm.at[wid, u], rowv.at[slot], sem)

    def _stage_wait(u, slot, sem):
        pltpu.make_async_copy(col_hbm.at[pl.ds(wid * EPW + u * CH, CH)],
                              colv.at[pl.ds(slot * CH, CH)], sem).wait()
        pltpu.make_async_copy(w_hbm.at[pl.ds(wid * EPW + u * CH, CH)],
                              wv.at[pl.ds(slot * CH, CH)], sem).wait()
        pltpu.make_async_copy(row_hbm.at[wid, u], rowv.at[slot], sem).wait()

    def _cidx(gb):
        return colv.at[pl.ds(((gb // SB) % 2) * CH + (gb % SB) * K, K)]

    def _gather(gb, rbuf, sem):
        pltpu.async_copy(ego_hbm.at[_cidx(gb)], rbuf, sem)

    def _gather_wait(gb, rbuf, sem):
        pltpu.make_async_copy(ego_hbm.at[_cidx(gb)], rbuf, sem).wait()

    def _scatter(gb, tbuf, sem):
        pltpu.async_copy(tbuf, acc.at[rowv.at[(gb // SB) % 2, gb % SB]],
                         sem, add=True)

    def _scatter_wait(gb, tbuf, sem):
        pltpu.make_async_copy(tbuf, acc.at[rowv.at[(gb // SB) % 2, gb % SB]],
                              sem).wait()

    def _scale(gb, rbuf, tbuf):
        base = ((gb // SB) % 2) * CH + (gb % SB) * K
        for g in range((K + 15) // 16):
            wvec16 = wv[pl.ds(base + g * 16, 16)]
            for e16 in range(min(16, K - g * 16)):
                wb = wvec16[jnp.full((16,), e16, jnp.int32)]
                e = g * 16 + e16
                for j in range(D // 16):
                    tbuf[e, pl.ds(j * 16, 16)] = (
                        rbuf[e, pl.ds(j * 16, 16)] * wb)

    _stage(0, 0, stg0)
    _stage_wait(0, 0, stg0)
    _gather(0, r0, g0)
    _gather(1, r1, g1)

    def _pair(gp, carry):
        gb0 = 2 * gp
        gb1 = gb0 + 1
        u = gb0 // SB
        cur = u % 2
        bl0 = gb0 % SB

        @pl.when(jnp.logical_and(bl0 == 2, u + 1 < NSC))
        def _prefetch():
            @pl.when(cur == 0)
            def _p0():
                _stage(u + 1, 1, stg1)

            @pl.when(cur == 1)
            def _p1():
                _stage(u + 1, 0, stg0)

        @pl.when(jnp.logical_and(bl0 == SB - 2, u + 1 < NSC))
        def _stgwait():
            @pl.when(cur == 0)
            def _w0():
                _stage_wait(u + 1, 1, stg1)

            @pl.when(cur == 1)
            def _w1():
                _stage_wait(u + 1, 0, stg0)

        _gather_wait(gb0, r0, g0)

        @pl.when(gp > 0)
        def _sw0():
            _scatter_wait(gb0, t0, s0)

        _scale(gb0, r0, t0)

        @pl.when(gp < NPAIR - 1)
        def _ng0():
            _gather(gb0 + 2, r0, g0)

        _scatter(gb0, t0, s0)

        _gather_wait(gb1, r1, g1)

        @pl.when(gp > 0)
        def _sw1():
            _scatter_wait(gb1, t1, s1)

        _scale(gb1, r1, t1)

        @pl.when(gp < NPAIR - 1)
        def _ng1():
            _gather(gb1 + 2, r1, g1)

        _scatter(gb1, t1, s1)
        return carry

    lax.fori_loop(0, NPAIR, _pair, 0)
    _scatter_wait(NB - 2, t0, s0)
    _scatter_wait(NB - 1, t1, s1)
    plsc.subcore_barrier()

    pltpu.sync_copy(acc.at[pl.ds(s * RPT, RPT)],
                    out_hbm.at[pl.ds(c * N + s * RPT, RPT)])

    @pl.when(s == NS - 1)
    def _out_rem():
        pltpu.sync_copy(acc.at[pl.ds(NS * RPT, REM)],
                        out_hbm.at[pl.ds(c * N + NS * RPT, REM)])


BLK = 2000


def _bi_interact(ego, side, w1_ref, b1_ref, w2_ref, b2_ref):
    h1 = jnp.dot(ego + side, w1_ref[...],
                 preferred_element_type=jnp.float32) + b1_ref[...]
    h1 = jnp.maximum(h1, 0.01 * h1)
    h2 = jnp.dot(ego * side, w2_ref[...],
                 preferred_element_type=jnp.float32) + b2_ref[...]
    h2 = jnp.maximum(h2, 0.01 * h2)
    e = h1 + h2
    ss = jnp.sum(e * e, axis=1, keepdims=True)
    nrm = e / jnp.maximum(jnp.sqrt(ss), 1e-12)
    return e, nrm


def _dense1_body(ego_ref, sa_ref, sb_ref, w1_ref, b1_ref, w2_ref, b2_ref,
                 ego_out_ref, nrm_out_ref):
    side = sa_ref[...] + sb_ref[...]
    e, nrm = _bi_interact(ego_ref[...], side, w1_ref, b1_ref, w2_ref, b2_ref)
    ego_out_ref[...] = e
    nrm_out_ref[...] = nrm


def _dense2_body(x_ref, n1_ref, ego_ref, sa_ref, sb_ref,
                 w1_ref, b1_ref, w2_ref, b2_ref, out_ref):
    side = sa_ref[...] + sb_ref[...]
    _, nrm = _bi_interact(ego_ref[...], side, w1_ref, b1_ref, w2_ref, b2_ref)
    out_ref[:, 0:D] = x_ref[...]
    out_ref[:, D:2 * D] = n1_ref[...]
    out_ref[:, 2 * D:3 * D] = nrm


_ROW_SPEC = pl.BlockSpec((BLK, D), lambda i: (i, 0))
_SHIFT_SPEC = pl.BlockSpec((BLK, D), lambda i: (i + N // BLK, 0))
_W_SPEC = pl.BlockSpec((D, D), lambda i: (0, 0))
_B_SPEC = pl.BlockSpec((1, D), lambda i: (0, 0))


def _dense1_tc(ego, side2, W1, b1, W2, b2):
    return pl.pallas_call(
        _dense1_body,
        grid=(N // BLK,),
        in_specs=[_ROW_SPEC, _ROW_SPEC, _SHIFT_SPEC,
                  _W_SPEC, _B_SPEC, _W_SPEC, _B_SPEC],
        out_specs=[_ROW_SPEC, _ROW_SPEC],
        out_shape=[
            jax.ShapeDtypeStruct((N, D), jnp.float32),
            jax.ShapeDtypeStruct((N, D), jnp.float32),
        ],
    )(ego, side2, side2, W1, b1, W2, b2)


def _dense2_tc(x, n1, ego, side2, W1, b1, W2, b2):
    return pl.pallas_call(
        _dense2_body,
        grid=(N // BLK,),
        in_specs=[_ROW_SPEC, _ROW_SPEC, _ROW_SPEC, _ROW_SPEC, _SHIFT_SPEC,
                  _W_SPEC, _B_SPEC, _W_SPEC, _B_SPEC],
        out_specs=[pl.BlockSpec((BLK, 3 * D), lambda i: (i, 0))],
        out_shape=[jax.ShapeDtypeStruct((N, 3 * D), jnp.float32)],
    )(x, n1, ego, side2, side2, W1, b1, W2, b2)


def kernel(x, edge_index, edge_weight, W1_0, b1_0, W2_0, b2_0,
           W1_1, b1_1, W2_1, b2_1):
    row = edge_index[0].astype(jnp.int32).reshape(NW, NSC, SB, K)
    col = edge_index[1].astype(jnp.int32)
    w = edge_weight.astype(jnp.float32)

    side2 = _side_sc(x, col, row, w)
    ego1, n1 = _dense1_tc(x, side2, W1_0, b1_0.reshape(1, D),
                          W2_0, b2_0.reshape(1, D))
    side2b = _side_sc(ego1, col, row, w)
    (out,) = _dense2_tc(x, n1, ego1, side2b, W1_1, b1_1.reshape(1, D),
                        W2_1, b2_1.reshape(1, D))
    return out
